# Initial kernel scaffold; baseline (speedup 1.0000x reference)
import jax
import jax.numpy as jnp
from jax.experimental import pallas as pl


def kernel(features, neighbor_idx, weight, bias):
    raise NotImplementedError("write your pallas kernel here")


# speedup vs baseline: 1.5672x; 1.5672x over previous
import functools

import jax
import jax.numpy as jnp
from jax import lax
from jax.experimental import pallas as pl
from jax.experimental.pallas import tpu as pltpu
from jax.experimental.pallas import tpu_sc as plsc

N = 10000
NIN = 512
NOUT = 512
FV = 13

_NC = 2
_NS = 16
_NW = _NC * _NS
_LANES = 16

NPAD = 10240
_ROWS_PER_W = NPAD // _NW
_CHUNK = 8
_IPC = _CHUNK * FV
_CHUNKS = _ROWS_PER_W // _CHUNK
_IDS_PER_W = _ROWS_PER_W * FV


def _mm_body(f_ref, w_ref, t_ref):
    t_ref[0] = jnp.dot(f_ref[...], w_ref[0], preferred_element_type=jnp.float32)


def _matmul_all_offsets(features, w):
    blk = 2000
    grid = (N // blk, FV)
    return pl.pallas_call(
        _mm_body,
        grid=grid,
        in_specs=[
            pl.BlockSpec((blk, NIN), lambda nb, f: (nb, 0)),
            pl.BlockSpec((1, NIN, NOUT), lambda nb, f: (f, 0, 0)),
        ],
        out_specs=pl.BlockSpec((1, blk, NOUT), lambda nb, f: (f, nb, 0)),
        out_shape=jax.ShapeDtypeStruct((FV, N, NOUT), jnp.float32),
        compiler_params=pltpu.CompilerParams(
            dimension_semantics=("parallel", "parallel"),
        ),
    )(features, w)


def _sc_body(t_hbm, ids_hbm, bias_hbm, out_hbm, ids_v, gbuf, outbuf, bias_v, sem):
    wid = lax.axis_index("s") * _NC + lax.axis_index("c")
    pltpu.sync_copy(ids_hbm.at[pl.ds(wid * _IDS_PER_W, _IDS_PER_W)], ids_v)
    pltpu.sync_copy(bias_hbm, bias_v)

    def chunk(ch, carry):
        pltpu.async_copy(
            t_hbm.at[ids_v.at[pl.ds(ch * _IPC, _IPC)]], gbuf, sem
        ).wait()

        def row(r, carry):
            def col(j, carry):
                acc = bias_v[pl.ds(j * _LANES, _LANES)]
                for f in range(FV):
                    acc = acc + gbuf[r * FV + f, pl.ds(j * _LANES, _LANES)]
                outbuf[r, pl.ds(j * _LANES, _LANES)] = acc
                return carry

            return lax.fori_loop(0, NOUT // _LANES, col, carry)

        lax.fori_loop(0, _CHUNK, row, carry)
        pltpu.sync_copy(
            outbuf, out_hbm.at[pl.ds(wid * _ROWS_PER_W + ch * _CHUNK, _CHUNK)]
        )
        return carry

    lax.fori_loop(0, _CHUNKS, chunk, 0)


def _sc_gather_sum(t_flat, ids_flat, bias):
    mesh = plsc.VectorSubcoreMesh(core_axis_name="c", subcore_axis_name="s")
    k = functools.partial(
        pl.kernel,
        out_type=jax.ShapeDtypeStruct((NPAD, NOUT), jnp.float32),
        mesh=mesh,
        scratch_types=[
            pltpu.VMEM((_IDS_PER_W,), jnp.int32),
            pltpu.VMEM((_IPC, NOUT), jnp.float32),
            pltpu.VMEM((_CHUNK, NOUT), jnp.float32),
            pltpu.VMEM((NOUT,), jnp.float32),
            pltpu.SemaphoreType.DMA,
        ],
    )(_sc_body)
    return k(t_flat, ids_flat, bias)


def kernel(features, neighbor_idx, weight, bias):
    w = weight[:, 0, :, :]
    t = _matmul_all_offsets(features, w)
    t_flat = t.reshape(FV * N, NOUT)
    ids = neighbor_idx.astype(jnp.int32) + (
        jnp.arange(FV, dtype=jnp.int32) * N
    )[None, :]
    ids = jnp.pad(ids, ((0, NPAD - N), (0, 0))).reshape(-1)
    out = _sc_gather_sum(t_flat, ids, bias)
    return out[:N]


# speedup vs baseline: 2.1500x; 1.3719x over previous
import functools

import jax
import jax.numpy as jnp
from jax import lax
from jax.experimental import pallas as pl
from jax.experimental.pallas import tpu as pltpu
from jax.experimental.pallas import tpu_sc as plsc

N = 10000
NIN = 512
NOUT = 512
FV = 13

_NC = 2
_NS = 16
_NW = _NC * _NS
_LANES = 16

NPAD = 10240
_ROWS_PER_W = NPAD // _NW
_CHUNK = 8
_IPC = _CHUNK * FV
_CHUNKS = _ROWS_PER_W // _CHUNK
_IDS_PER_W = _ROWS_PER_W * FV


def _mm_body(f_ref, w_ref, t_ref):
    t_ref[0] = jnp.dot(f_ref[...], w_ref[0], preferred_element_type=jnp.float32)


def _matmul_all_offsets(features, w):
    blk = 2000
    grid = (N // blk, FV)
    return pl.pallas_call(
        _mm_body,
        grid=grid,
        in_specs=[
            pl.BlockSpec((blk, NIN), lambda nb, f: (nb, 0)),
            pl.BlockSpec((1, NIN, NOUT), lambda nb, f: (f, 0, 0)),
        ],
        out_specs=pl.BlockSpec((1, blk, NOUT), lambda nb, f: (f, nb, 0)),
        out_shape=jax.ShapeDtypeStruct((FV, N, NOUT), jnp.float32),
        compiler_params=pltpu.CompilerParams(
            dimension_semantics=("parallel", "parallel"),
        ),
    )(features, w)


def _sc_body(t_hbm, ids_hbm, bias_hbm, out_hbm, ids_v, g0, g1, outbuf, bias_v,
             s0, s1):
    wid = lax.axis_index("s") * _NC + lax.axis_index("c")
    pltpu.sync_copy(ids_hbm.at[pl.ds(wid * _IDS_PER_W, _IDS_PER_W)], ids_v)
    pltpu.sync_copy(bias_hbm, bias_v)

    def issue(ch, buf, sem):
        pltpu.async_copy(t_hbm.at[ids_v.at[pl.ds(ch * _IPC, _IPC)]], buf, sem)

    def wait(buf, sem):
        pltpu.make_async_copy(
            t_hbm.at[ids_v.at[pl.ds(0, _IPC)]], buf, sem
        ).wait()

    def consume(ch, buf):
        def row(r, carry):
            def col(j, carry):
                acc = bias_v[pl.ds(j * _LANES, _LANES)]
                for f in range(FV):
                    acc = acc + buf[r * FV + f, pl.ds(j * _LANES, _LANES)]
                outbuf[r, pl.ds(j * _LANES, _LANES)] = acc
                return carry

            return lax.fori_loop(0, NOUT // _LANES, col, carry)

        lax.fori_loop(0, _CHUNK, row, 0)
        pltpu.sync_copy(
            outbuf, out_hbm.at[pl.ds(wid * _ROWS_PER_W + ch * _CHUNK, _CHUNK)]
        )

    issue(0, g0, s0)
    issue(1, g1, s1)

    def steady(i, carry):
        ch = 2 * i
        wait(g0, s0)
        consume(ch, g0)
        issue(ch + 2, g0, s0)
        wait(g1, s1)
        consume(ch + 1, g1)
        issue(ch + 3, g1, s1)
        return carry

    lax.fori_loop(0, _CHUNKS // 2 - 1, steady, 0)
    wait(g0, s0)
    consume(_CHUNKS - 2, g0)
    wait(g1, s1)
    consume(_CHUNKS - 1, g1)


def _sc_gather_sum(t_flat, ids_flat, bias):
    mesh = plsc.VectorSubcoreMesh(core_axis_name="c", subcore_axis_name="s")
    k = functools.partial(
        pl.kernel,
        out_type=jax.ShapeDtypeStruct((NPAD, NOUT), jnp.float32),
        mesh=mesh,
        scratch_types=[
            pltpu.VMEM((_IDS_PER_W,), jnp.int32),
            pltpu.VMEM((_IPC, NOUT), jnp.float32),
            pltpu.VMEM((_IPC, NOUT), jnp.float32),
            pltpu.VMEM((_CHUNK, NOUT), jnp.float32),
            pltpu.VMEM((NOUT,), jnp.float32),
            pltpu.SemaphoreType.DMA,
            pltpu.SemaphoreType.DMA,
        ],
    )(_sc_body)
    return k(t_flat, ids_flat, bias)


def kernel(features, neighbor_idx, weight, bias):
    w = weight[:, 0, :, :]
    t = _matmul_all_offsets(features, w)
    t_flat = t.reshape(FV * N, NOUT)
    ids = neighbor_idx.astype(jnp.int32) + (
        jnp.arange(FV, dtype=jnp.int32) * N
    )[None, :]
    ids = jnp.pad(ids, ((0, NPAD - N), (0, 0))).reshape(-1)
    out = _sc_gather_sum(t_flat, ids, bias)
    return out[:N]


# speedup vs baseline: 2.5747x; 1.1975x over previous
import functools

import numpy as np

import jax
import jax.numpy as jnp
from jax import lax
from jax.experimental import pallas as pl
from jax.experimental.pallas import tpu as pltpu
from jax.experimental.pallas import tpu_sc as plsc

N = 10000
NIN = 512
NOUT = 512
FV = 13
_H = NOUT // 2

_NC = 2
_NS = 16
_NW = _NC * _NS
_LANES = 16

NPAD = 10240
_ROWS_PER_W = NPAD // _NW
_CHUNK = 8
_IPC = _CHUNK * FV
_CHUNKS = _ROWS_PER_W // _CHUNK
_IDS_PER_W = _ROWS_PER_W * FV

_HI = np.uint32(0xFFFF0000)


def _pack_words(y):
    a = y[..., :_H].astype(jnp.bfloat16).astype(jnp.float32)
    b = y[..., _H:].astype(jnp.bfloat16).astype(jnp.float32)
    au = lax.bitcast_convert_type(a, jnp.uint32)
    bu = lax.bitcast_convert_type(b, jnp.uint32)
    return lax.bitcast_convert_type((au >> 16) | (bu & _HI), jnp.int32)


def _mm_body(f_ref, w_ref, t_ref):
    y = jnp.dot(f_ref[...], w_ref[0], preferred_element_type=jnp.float32)
    t_ref[0] = _pack_words(y)


def _matmul_all_offsets(features, w):
    blk = 2000
    grid = (N // blk, FV)
    return pl.pallas_call(
        _mm_body,
        grid=grid,
        in_specs=[
            pl.BlockSpec((blk, NIN), lambda nb, f: (nb, 0)),
            pl.BlockSpec((1, NIN, NOUT), lambda nb, f: (f, 0, 0)),
        ],
        out_specs=pl.BlockSpec((1, blk, _H), lambda nb, f: (f, nb, 0)),
        out_shape=jax.ShapeDtypeStruct((FV, N, _H), jnp.int32),
        compiler_params=pltpu.CompilerParams(
            dimension_semantics=("parallel", "parallel"),
        ),
    )(features, w)


def _sc_body(t_hbm, ids_hbm, bias_hbm, out_hbm, ids_v, g0, g1, outbuf, bias_v,
             s0, s1):
    wid = lax.axis_index("s") * _NC + lax.axis_index("c")
    pltpu.sync_copy(ids_hbm.at[pl.ds(wid * _IDS_PER_W, _IDS_PER_W)], ids_v)
    pltpu.sync_copy(bias_hbm, bias_v)

    def issue(ch, buf, sem):
        pltpu.async_copy(t_hbm.at[ids_v.at[pl.ds(ch * _IPC, _IPC)]], buf, sem)

    def wait(buf, sem):
        pltpu.make_async_copy(
            t_hbm.at[ids_v.at[pl.ds(0, _IPC)]], buf, sem
        ).wait()

    def consume(ch, buf):
        def row(r, carry):
            for g in range(_H // _LANES):
                sl = pl.ds(g * _LANES, _LANES)
                acc_lo = bias_v[sl]
                acc_hi = bias_v[pl.ds(_H + g * _LANES, _LANES)]
                for f in range(FV):
                    w = buf[r * FV + f, sl]
                    lo = lax.bitcast_convert_type(
                        lax.shift_left(w, 16), jnp.float32)
                    hi = lax.bitcast_convert_type(
                        lax.bitwise_and(w, jnp.int32(-65536)), jnp.float32)
                    acc_lo = acc_lo + lo
                    acc_hi = acc_hi + hi
                outbuf[r, sl] = acc_lo
                outbuf[r, pl.ds(_H + g * _LANES, _LANES)] = acc_hi
            return carry

        lax.fori_loop(0, _CHUNK, row, 0)
        pltpu.sync_copy(
            outbuf, out_hbm.at[pl.ds(wid * _ROWS_PER_W + ch * _CHUNK, _CHUNK)]
        )

    issue(0, g0, s0)
    issue(1, g1, s1)

    def steady(i, carry):
        ch = 2 * i
        wait(g0, s0)
        consume(ch, g0)
        issue(ch + 2, g0, s0)
        wait(g1, s1)
        consume(ch + 1, g1)
        issue(ch + 3, g1, s1)
        return carry

    lax.fori_loop(0, _CHUNKS // 2 - 1, steady, 0)
    wait(g0, s0)
    consume(_CHUNKS - 2, g0)
    wait(g1, s1)
    consume(_CHUNKS - 1, g1)


def _sc_gather_sum(t2, ids_flat, bias):
    mesh = plsc.VectorSubcoreMesh(core_axis_name="c", subcore_axis_name="s")
    k = functools.partial(
        pl.kernel,
        out_type=jax.ShapeDtypeStruct((NPAD, NOUT), jnp.float32),
        mesh=mesh,
        scratch_types=[
            pltpu.VMEM((_IDS_PER_W,), jnp.int32),
            pltpu.VMEM((_IPC, _H), jnp.int32),
            pltpu.VMEM((_IPC, _H), jnp.int32),
            pltpu.VMEM((_CHUNK, NOUT), jnp.float32),
            pltpu.VMEM((NOUT,), jnp.float32),
            pltpu.SemaphoreType.DMA,
            pltpu.SemaphoreType.DMA,
        ],
    )(_sc_body)
    return k(t2, ids_flat, bias)


def kernel(features, neighbor_idx, weight, bias):
    w = weight[:, 0, :, :].astype(jnp.bfloat16)
    f_bf = features.astype(jnp.bfloat16)
    t = _matmul_all_offsets(f_bf, w)
    t2 = t.reshape(FV * N, _H)
    ids = neighbor_idx.astype(jnp.int32) + (
        jnp.arange(FV, dtype=jnp.int32) * N
    )[None, :]
    ids = jnp.pad(ids, ((0, NPAD - N), (0, 0))).reshape(-1)
    out = _sc_gather_sum(t2, ids, bias)
    return out[:N]
